# baseline (speedup 1.0000x reference)
import functools

import jax
import jax.numpy as jnp
from jax import lax
from jax.experimental import pallas as pl
from jax.experimental.pallas import tpu as pltpu
from jax.experimental.pallas import tpu_sc as plsc

_NC = 2
_NS = 16
_NW = _NC * _NS
_CHR = 256
_NB = 4


def _make_sc_copy(rows, d):
    rows_per_w = rows // _NW
    nj_per_tensor = rows_per_w // _CHR
    mesh = plsc.VectorSubcoreMesh(
        core_axis_name="c", subcore_axis_name="s",
        num_cores=_NC, num_subcores=_NS)

    @functools.partial(
        pl.kernel,
        out_type=[jax.ShapeDtypeStruct((rows, d), jnp.float16)] * 2,
        mesh=mesh,
        scratch_types=(
            [pltpu.VMEM((_CHR, d), jnp.float16)] * _NB
            + [pltpu.SemaphoreType.DMA] * (2 * _NB)
        ),
    )
    def sc_copy(k_hbm, v_hbm, ko_hbm, vo_hbm,
                b0, b1, b2, b3, si0, si1, si2, si3, so0, so1, so2, so3):
        bufs = (b0, b1, b2, b3)
        sin = (si0, si1, si2, si3)
        sout = (so0, so1, so2, so3)
        wid = lax.axis_index("s") * _NC + lax.axis_index("c")
        base = wid * rows_per_w

        jobs = []
        for src, dst in ((k_hbm, ko_hbm), (v_hbm, vo_hbm)):
            for c in range(nj_per_tensor):
                jobs.append((src, dst, c * _CHR))
        ins, outs = [], []
        for j, (src, dst, off) in enumerate(jobs):
            b = j % _NB
            sl = pl.ds(base + off, _CHR)
            ins.append(pltpu.make_async_copy(src.at[sl], bufs[b], sin[b]))
            outs.append(pltpu.make_async_copy(bufs[b], dst.at[sl], sout[b]))

        nj = len(jobs)
        for j in range(min(_NB, nj)):
            ins[j].start()
        for j in range(nj):
            ins[j].wait()
            outs[j].start()
            nxt = j + _NB
            if nxt < nj:
                outs[j].wait()
                ins[nxt].start()
        for j in range(max(0, nj - _NB), nj):
            outs[j].wait()

    return sc_copy


def kernel(key_states, value_states, k_cache, v_cache, layer_idx):
    B, H, S, D = key_states.shape
    rows = B * H * S
    k = key_states.reshape(rows, D)
    v = value_states.reshape(rows, D)
    ko, vo = _make_sc_copy(rows, D)(k, v)
    return ko.reshape(B, H, S, D), vo.reshape(B, H, S, D)


# speedup vs baseline: 1.0019x; 1.0019x over previous
import functools

import jax
import jax.numpy as jnp
from jax import lax
from jax.experimental import pallas as pl
from jax.experimental.pallas import tpu as pltpu
from jax.experimental.pallas import tpu_sc as plsc

_NC = 2
_NS = 16
_NW = _NC * _NS
_CHR = 512
_NB = 2


def _make_sc_copy(rows, d):
    rows_per_w = rows // _NW
    nj_per_tensor = rows_per_w // _CHR
    mesh = plsc.VectorSubcoreMesh(
        core_axis_name="c", subcore_axis_name="s",
        num_cores=_NC, num_subcores=_NS)

    @functools.partial(
        pl.kernel,
        out_type=[jax.ShapeDtypeStruct((rows, d), jnp.float16)] * 2,
        mesh=mesh,
        scratch_types=(
            [pltpu.VMEM((_CHR, d), jnp.float16)] * _NB
            + [pltpu.SemaphoreType.DMA] * (2 * _NB)
        ),
    )
    def sc_copy(k_hbm, v_hbm, ko_hbm, vo_hbm,
                b0, b1, si0, si1, so0, so1):
        bufs = (b0, b1)
        sin = (si0, si1)
        sout = (so0, so1)
        wid = lax.axis_index("s") * _NC + lax.axis_index("c")
        base = wid * rows_per_w

        jobs = []
        for src, dst in ((k_hbm, ko_hbm), (v_hbm, vo_hbm)):
            for c in range(nj_per_tensor):
                jobs.append((src, dst, c * _CHR))
        ins, outs = [], []
        for j, (src, dst, off) in enumerate(jobs):
            b = j % _NB
            sl = pl.ds(base + off, _CHR)
            ins.append(pltpu.make_async_copy(src.at[sl], bufs[b], sin[b]))
            outs.append(pltpu.make_async_copy(bufs[b], dst.at[sl], sout[b]))

        nj = len(jobs)
        for j in range(min(_NB, nj)):
            ins[j].start()
        for j in range(nj):
            ins[j].wait()
            outs[j].start()
            nxt = j + _NB
            if nxt < nj:
                outs[j].wait()
                ins[nxt].start()
        for j in range(max(0, nj - _NB), nj):
            outs[j].wait()

    return sc_copy


def kernel(key_states, value_states, k_cache, v_cache, layer_idx):
    B, H, S, D = key_states.shape
    rows = B * H * S
    k = key_states.reshape(rows, D)
    v = value_states.reshape(rows, D)
    ko, vo = _make_sc_copy(rows, D)(k, v)
    return ko.reshape(B, H, S, D), vo.reshape(B, H, S, D)


# speedup vs baseline: 1.0239x; 1.0220x over previous
import functools

import jax
import jax.numpy as jnp
from jax import lax
from jax.experimental import pallas as pl
from jax.experimental.pallas import tpu as pltpu
from jax.experimental.pallas import tpu_sc as plsc

_NC = 2
_NS = 16
_NW = _NC * _NS
_CHR = 512
_NB = 3


def _make_sc_copy(rows, d):
    rows_per_w = rows // _NW
    nj_per_tensor = rows_per_w // _CHR
    mesh = plsc.VectorSubcoreMesh(
        core_axis_name="c", subcore_axis_name="s",
        num_cores=_NC, num_subcores=_NS)

    @functools.partial(
        pl.kernel,
        out_type=[jax.ShapeDtypeStruct((rows, d), jnp.float16)] * 2,
        mesh=mesh,
        scratch_types=(
            [pltpu.VMEM((_CHR, d), jnp.float16)] * _NB
            + [pltpu.SemaphoreType.DMA] * (2 * _NB)
        ),
    )
    def sc_copy(k_hbm, v_hbm, ko_hbm, vo_hbm,
                b0, b1, b2, si0, si1, si2, so0, so1, so2):
        bufs = (b0, b1, b2)
        sin = (si0, si1, si2)
        sout = (so0, so1, so2)
        wid = lax.axis_index("s") * _NC + lax.axis_index("c")
        base = wid * rows_per_w

        jobs = []
        for src, dst in ((k_hbm, ko_hbm), (v_hbm, vo_hbm)):
            for c in range(nj_per_tensor):
                jobs.append((src, dst, c * _CHR))
        ins, outs = [], []
        for j, (src, dst, off) in enumerate(jobs):
            b = j % _NB
            sl = pl.ds(base + off, _CHR)
            ins.append(pltpu.make_async_copy(src.at[sl], bufs[b], sin[b]))
            outs.append(pltpu.make_async_copy(bufs[b], dst.at[sl], sout[b]))

        nj = len(jobs)
        for j in range(min(_NB, nj)):
            ins[j].start()
        for j in range(nj):
            ins[j].wait()
            outs[j].start()
            nxt = j + _NB
            if nxt < nj:
                outs[j].wait()
                ins[nxt].start()
        for j in range(max(0, nj - _NB), nj):
            outs[j].wait()

    return sc_copy


def kernel(key_states, value_states, k_cache, v_cache, layer_idx):
    B, H, S, D = key_states.shape
    rows = B * H * S
    k = key_states.reshape(rows, D)
    v = value_states.reshape(rows, D)
    ko, vo = _make_sc_copy(rows, D)(k, v)
    return ko.reshape(B, H, S, D), vo.reshape(B, H, S, D)


# speedup vs baseline: 1.0445x; 1.0201x over previous
import functools

import jax
import jax.numpy as jnp
from jax import lax
from jax.experimental import pallas as pl
from jax.experimental.pallas import tpu as pltpu
from jax.experimental.pallas import tpu_sc as plsc

_NC = 2
_NS = 16
_NW = _NC * _NS
_CHR = 512


def _make_sc_copy(rows, d):
    rows_per_w = rows // _NW
    nj_per_tensor = rows_per_w // _CHR
    half = nj_per_tensor // 2
    mesh = plsc.VectorSubcoreMesh(
        core_axis_name="c", subcore_axis_name="s",
        num_cores=_NC, num_subcores=_NS)

    @functools.partial(
        pl.kernel,
        out_type=[jax.ShapeDtypeStruct((rows, d), jnp.float16)] * 2,
        mesh=mesh,
        scratch_types=(
            [pltpu.VMEM((_CHR, d), jnp.float16)] * 2
            + [pltpu.MemorySpace.VMEM_SHARED((_NS, 2, _CHR, d), jnp.float16)]
            + [pltpu.SemaphoreType.DMA] * 8
        ),
    )
    def sc_copy(k_hbm, v_hbm, ko_hbm, vo_hbm,
                t0, t1, sh, ai0, ai1, ao0, ao1, bi0, bi1, bo0, bo1):
        sid = lax.axis_index("s")
        wid = sid * _NC + lax.axis_index("c")
        base = wid * rows_per_w

        abufs, asin, asout = (t0, t1), (ai0, ai1), (ao0, ao1)
        bbufs = (sh.at[sid, 0], sh.at[sid, 1])
        bsin, bsout = (bi0, bi1), (bo0, bo1)

        ajobs, bjobs = [], []
        for src, dst in ((k_hbm, ko_hbm), (v_hbm, vo_hbm)):
            for c in range(nj_per_tensor):
                (ajobs if c < half else bjobs).append((src, dst, c * _CHR))

        def mk(jobs, bufs, sin, sout):
            ins, outs = [], []
            for j, (src, dst, off) in enumerate(jobs):
                b = j % 2
                sl = pl.ds(base + off, _CHR)
                ins.append(pltpu.make_async_copy(src.at[sl], bufs[b], sin[b]))
                outs.append(pltpu.make_async_copy(bufs[b], dst.at[sl], sout[b]))
            return ins, outs

        ains, aouts = mk(ajobs, abufs, asin, asout)
        bins, bouts = mk(bjobs, bbufs, bsin, bsout)

        nj = len(ajobs)
        for j in range(min(2, nj)):
            ains[j].start()
            bins[j].start()
        for j in range(nj):
            ains[j].wait()
            aouts[j].start()
            bins[j].wait()
            bouts[j].start()
            nxt = j + 2
            if nxt < nj:
                aouts[j].wait()
                ains[nxt].start()
                bouts[j].wait()
                bins[nxt].start()
        for j in range(max(0, nj - 2), nj):
            aouts[j].wait()
            bouts[j].wait()

    return sc_copy


def kernel(key_states, value_states, k_cache, v_cache, layer_idx):
    B, H, S, D = key_states.shape
    rows = B * H * S
    k = key_states.reshape(rows, D)
    v = value_states.reshape(rows, D)
    ko, vo = _make_sc_copy(rows, D)(k, v)
    return ko.reshape(B, H, S, D), vo.reshape(B, H, S, D)
